# Initial kernel scaffold; baseline (speedup 1.0000x reference)
import jax
import jax.numpy as jnp
from jax import lax
from jax.experimental import pallas as pl
from jax.experimental.pallas import tpu as pltpu

_B = 16384
_V = 33
_DIN = 16
_DOUT = 128
_R = 1024
_G = _B // _R
_HK = 128
_GK = 128


def _body(inp_ref, L0_ref, fs_ref, b_ref, vut_ref, hpat_ref,
          out_ref, hold_ref, hg_ref, lh_ref):
    Lrow = L0_ref[...]
    m = jnp.max(Lrow, axis=1, keepdims=True)
    e = jnp.exp(Lrow - m)
    w = e / jnp.sum(e, axis=1, keepdims=True)
    fc = lax.dot_general(w, fs_ref[...], (((1,), (0,)), ((), ())),
                         preferred_element_type=jnp.float32) + b_ref[...]
    gpat = lax.dot_general(w, vut_ref[...], (((1,), (0,)), ((), ())),
                           preferred_element_type=jnp.float32)
    rw = 1.0 / (w + 1e-20)

    x = inp_ref[...]
    s = jnp.sum(x, axis=1, keepdims=True)
    iw = jnp.concatenate([1.0 - s, x], axis=1)
    wd = iw * rw
    val = jnp.min(wd, axis=1, keepdims=True)
    iota = lax.broadcasted_iota(jnp.int32, (_R, _V), 1)
    idx = jnp.min(jnp.where(wd == val, iota, _V), axis=1, keepdims=True)
    M = jnp.where(iota == idx, 0.0, iw - val * w)
    out = lax.dot_general(M, fs_ref[...], (((1,), (0,)), ((), ())),
                          preferred_element_type=jnp.float32)
    out_ref[...] = out + val * fc

    hold_ref[...] = jnp.broadcast_to(hpat_ref[...][None], (_HK, _V, 128))
    hg_ref[...] = jnp.broadcast_to(gpat, (_GK, 128))
    lh_ref[...] = jnp.broadcast_to(gpat, (_GK, 128))


def kernel(inp, L0, visible_fs, biases, visible_units):
    hpat = jnp.tile(visible_units.reshape(-1), 8).reshape(_V, 128)
    vut = jnp.tile(visible_units, (1, 128 // _DIN))

    out128, hold3, hg, lh = pl.pallas_call(
        _body,
        grid=(_G,),
        in_specs=[
            pl.BlockSpec((_R, _V - 1), lambda i: (i, 0)),
            pl.BlockSpec((1, _V), lambda i: (0, 0)),
            pl.BlockSpec((_V, _DOUT), lambda i: (0, 0)),
            pl.BlockSpec((1, _DOUT), lambda i: (0, 0)),
            pl.BlockSpec((_V, 128), lambda i: (0, 0)),
            pl.BlockSpec((_V, 128), lambda i: (0, 0)),
        ],
        out_specs=[
            pl.BlockSpec((_R, _DOUT), lambda i: (i, 0)),
            pl.BlockSpec((_HK, _V, 128), lambda i: (i, 0, 0)),
            pl.BlockSpec((_GK, 128), lambda i: (i, 0)),
            pl.BlockSpec((_GK, 128), lambda i: (i, 0)),
        ],
        out_shape=[
            jax.ShapeDtypeStruct((_B, _DOUT), jnp.float32),
            jax.ShapeDtypeStruct((_G * _HK, _V, 128), jnp.float32),
            jax.ShapeDtypeStruct((_G * _GK, 128), jnp.float32),
            jax.ShapeDtypeStruct((_G * _GK, 128), jnp.float32),
        ],
    )(inp, L0, visible_fs, biases, vut, hpat)

    return (out128.reshape(_B, 1, _DOUT),
            hold3.reshape(_B, _V, _DIN),
            hg.reshape(_B, _DIN),
            lh.reshape(_B, _DIN))


# speedup vs baseline: 12.9341x; 12.9341x over previous
import jax
import jax.numpy as jnp
from jax import lax
from jax.experimental import pallas as pl
from jax.experimental.pallas import tpu as pltpu

_B = 16384
_V = 33
_DIN = 16
_DOUT = 128
_R = 1024
_G = _B // _R
_HK = 128
_GK = 128


def _body(inp_ref, L0_ref, fs_ref, b_ref, vut_ref, hpat_ref,
          out_ref, hold_ref, hg_ref, lh_ref):
    Lrow = L0_ref[...]
    m = jnp.max(Lrow, axis=1, keepdims=True)
    e = jnp.exp(Lrow - m)
    w = e / jnp.sum(e, axis=1, keepdims=True)
    fc = lax.dot_general(w, fs_ref[...], (((1,), (0,)), ((), ())),
                         preferred_element_type=jnp.float32) + b_ref[...]
    gpat = lax.dot_general(w, vut_ref[...], (((1,), (0,)), ((), ())),
                           preferred_element_type=jnp.float32)
    x = inp_ref[...]
    s = jnp.sum(x, axis=1, keepdims=True)
    iw = jnp.concatenate([1.0 - s, x], axis=1)
    wd = iw / (w + 1e-20)
    val = jnp.min(wd, axis=1, keepdims=True)
    iota = lax.broadcasted_iota(jnp.int32, (_R, _V), 1)
    idx = jnp.min(jnp.where(wd == val, iota, _V), axis=1, keepdims=True)
    M = jnp.where(iota == idx, 0.0, iw - val * w)
    out = lax.dot_general(M, fs_ref[...], (((1,), (0,)), ((), ())),
                          preferred_element_type=jnp.float32)
    out_ref[...] = out + val * fc

    hold_ref[...] = jnp.broadcast_to(hpat_ref[...][None], (_HK, _V, 128))
    hg_ref[...] = jnp.broadcast_to(gpat, (_GK, 128))
    lh_ref[...] = jnp.broadcast_to(gpat, (_GK, 128))


def kernel(inp, L0, visible_fs, biases, visible_units):
    hpat = jnp.tile(visible_units.reshape(-1), 8).reshape(_V, 128)
    vut = jnp.tile(visible_units, (1, 128 // _DIN))

    out128, hold3, hg, lh = pl.pallas_call(
        _body,
        grid=(_G,),
        in_specs=[
            pl.BlockSpec((_R, _V - 1), lambda i: (i, 0)),
            pl.BlockSpec((1, _V), lambda i: (0, 0)),
            pl.BlockSpec((_V, _DOUT), lambda i: (0, 0)),
            pl.BlockSpec((1, _DOUT), lambda i: (0, 0)),
            pl.BlockSpec((_V, 128), lambda i: (0, 0)),
            pl.BlockSpec((_V, 128), lambda i: (0, 0)),
        ],
        out_specs=[
            pl.BlockSpec((_R, _DOUT), lambda i: (i, 0)),
            pl.BlockSpec((_HK, _V, 128), lambda i: (i, 0, 0)),
            pl.BlockSpec((_GK, 128), lambda i: (i, 0)),
            pl.BlockSpec((_GK, 128), lambda i: (i, 0)),
        ],
        out_shape=[
            jax.ShapeDtypeStruct((_B, _DOUT), jnp.float32),
            jax.ShapeDtypeStruct((_G * _HK, _V, 128), jnp.float32),
            jax.ShapeDtypeStruct((_G * _GK, 128), jnp.float32),
            jax.ShapeDtypeStruct((_G * _GK, 128), jnp.float32),
        ],
    )(inp, L0, visible_fs, biases, vut, hpat)

    return (out128.reshape(_B, 1, _DOUT),
            hold3.reshape(_B, _V, _DIN),
            hg.reshape(_B, _DIN),
            lh.reshape(_B, _DIN))


# speedup vs baseline: 16.1876x; 1.2515x over previous
import jax
import jax.numpy as jnp
from jax import lax
from jax.experimental import pallas as pl
from jax.experimental.pallas import tpu as pltpu

_B = 16384
_V = 33
_DIN = 16
_DOUT = 128
_R = 256
_G = _B // _R


def _body(inp_ref, L0_ref, fs_ref, b_ref, vu_ref,
          out_ref, hold_ref, hg_ref, lh_ref):
    Lrow = L0_ref[...]
    m = jnp.max(Lrow, axis=1, keepdims=True)
    e = jnp.exp(Lrow - m)
    w = e / jnp.sum(e, axis=1, keepdims=True)
    fc = lax.dot_general(w, fs_ref[...], (((1,), (0,)), ((), ())),
                         preferred_element_type=jnp.float32) + b_ref[...]
    hc = lax.dot_general(w, vu_ref[...], (((1,), (0,)), ((), ())),
                         preferred_element_type=jnp.float32)

    x = inp_ref[...]
    s = jnp.sum(x, axis=1, keepdims=True)
    iw = jnp.concatenate([1.0 - s, x], axis=1)
    wd = iw / (w + 1e-20)
    val = jnp.min(wd, axis=1, keepdims=True)
    iota = lax.broadcasted_iota(jnp.int32, (_R, _V), 1)
    idx = jnp.min(jnp.where(wd == val, iota, _V), axis=1, keepdims=True)
    M = jnp.where(iota == idx, 0.0, iw - val * w)
    out = lax.dot_general(M, fs_ref[...], (((1,), (0,)), ((), ())),
                          preferred_element_type=jnp.float32)
    out_ref[...] = (out + val * fc)[:, None, :]

    hold_ref[...] = jnp.broadcast_to(vu_ref[...][None], (_R, _V, _DIN))
    hg_ref[...] = jnp.broadcast_to(hc, (_R, _DIN))
    lh_ref[...] = jnp.broadcast_to(hc, (_R, _DIN))


def kernel(inp, L0, visible_fs, biases, visible_units):
    out, hold, hg, lh = pl.pallas_call(
        _body,
        grid=(_G,),
        in_specs=[
            pl.BlockSpec((_R, _V - 1), lambda i: (i, 0)),
            pl.BlockSpec((1, _V), lambda i: (0, 0)),
            pl.BlockSpec((_V, _DOUT), lambda i: (0, 0)),
            pl.BlockSpec((1, _DOUT), lambda i: (0, 0)),
            pl.BlockSpec((_V, _DIN), lambda i: (0, 0)),
        ],
        out_specs=[
            pl.BlockSpec((_R, 1, _DOUT), lambda i: (i, 0, 0)),
            pl.BlockSpec((_R, _V, _DIN), lambda i: (i, 0, 0)),
            pl.BlockSpec((_R, _DIN), lambda i: (i, 0)),
            pl.BlockSpec((_R, _DIN), lambda i: (i, 0)),
        ],
        out_shape=[
            jax.ShapeDtypeStruct((_B, 1, _DOUT), jnp.float32),
            jax.ShapeDtypeStruct((_B, _V, _DIN), jnp.float32),
            jax.ShapeDtypeStruct((_B, _DIN), jnp.float32),
            jax.ShapeDtypeStruct((_B, _DIN), jnp.float32),
        ],
    )(inp, L0, visible_fs, biases, visible_units)

    return (out, hold, hg, lh)
